# baseline (speedup 1.0000x reference)
import functools

import jax
import jax.numpy as jnp
from jax.experimental import pallas as pl
from jax.experimental.pallas import tpu as pltpu

_ALPHA = 0.25
_GAMMA = 2.0
_DELTA = 0.1


def _loss_body(tc_ref, tb_ref, gb_ref, to_ref, gc_ref, go_ref,
               cls_ref, obj_ref, bb_ref):
    i = pl.program_id(0)

    @pl.when(i == 0)
    def _():
        cls_ref[0, 0] = 0.0
        obj_ref[0, 0] = 0.0
        bb_ref[0, 0] = 0.0

    go = go_ref[...]
    gc = gc_ref[...]
    mask_obj = (go != -1).astype(jnp.float32)
    mask_bb = (go == 1).astype(jnp.float32)

    x = tc_ref[...]
    m = jnp.max(x, axis=1, keepdims=True)
    e = jnp.exp(x - m)
    s = jnp.sum(e, axis=1, keepdims=True)
    cls_ids = jax.lax.broadcasted_iota(jnp.int32, x.shape, 1)
    sel = jnp.sum(jnp.where(cls_ids == gc, x, 0.0), axis=1, keepdims=True)
    ce = (m - sel) + jnp.log(s)
    p = jnp.exp(-ce)
    focal_cls = _ALPHA * (1.0 - p) ** _GAMMA * ce
    cls_ref[0, 0] += jnp.sum(focal_cls * mask_bb)

    t = to_ref[...]
    a = t[:, 0:1]
    b = t[:, 1:2]
    m2 = jnp.maximum(a, b)
    s2 = jnp.exp(a - m2) + jnp.exp(b - m2)
    label = jnp.clip(go, 0, 1)
    sel2 = jnp.where(label == 1, b, a)
    ce2 = (m2 - sel2) + jnp.log(s2)
    p2 = jnp.exp(-ce2)
    focal_obj = _ALPHA * (1.0 - p2) ** _GAMMA * ce2
    obj_ref[0, 0] += jnp.sum(focal_obj * mask_obj)

    d = tb_ref[...] - gb_ref[...]
    ad = jnp.abs(d)
    sl1 = jnp.where(ad < _DELTA, 0.5 * d * d / _DELTA, ad - 0.5 * _DELTA)
    bb_ref[0, 0] += jnp.sum(sl1 * mask_bb)


@functools.partial(jax.jit, static_argnames=("interpret",))
def _loss_sums(tc2, tb2, gb2, to2, gc2, go2, interpret=False):
    n, c = tc2.shape
    rb = 2048
    nb = n // rb
    scalar_spec = pl.BlockSpec((1, 1), lambda i: (0, 0),
                               memory_space=pltpu.SMEM)
    return pl.pallas_call(
        _loss_body,
        grid=(nb,),
        in_specs=[
            pl.BlockSpec((rb, c), lambda i: (i, 0)),
            pl.BlockSpec((rb, 4), lambda i: (i, 0)),
            pl.BlockSpec((rb, 4), lambda i: (i, 0)),
            pl.BlockSpec((rb, 2), lambda i: (i, 0)),
            pl.BlockSpec((rb, 1), lambda i: (i, 0)),
            pl.BlockSpec((rb, 1), lambda i: (i, 0)),
        ],
        out_specs=[scalar_spec, scalar_spec, scalar_spec],
        out_shape=[jax.ShapeDtypeStruct((1, 1), jnp.float32)] * 3,
        compiler_params=pltpu.CompilerParams(
            dimension_semantics=("arbitrary",)),
        interpret=interpret,
    )(tc2, tb2, gb2, to2, gc2, go2)


def kernel(targets_bb, targets_cls, targets_obj, gt_targets_bb,
           gt_targets_cls, gt_targets_obj, w_obj, w_cls, w_bb, step,
           interpret=False):
    c = targets_cls.shape[-1]
    tc2 = jnp.reshape(targets_cls, (-1, c))
    tb2 = jnp.reshape(targets_bb, (-1, 4))
    gb2 = jnp.reshape(gt_targets_bb, (-1, 4))
    to2 = jnp.reshape(targets_obj, (-1, 2))
    gc2 = jnp.reshape(gt_targets_cls, (-1, 1))
    go2 = jnp.reshape(gt_targets_obj, (-1, 1))
    n = tc2.shape[0]

    cls_s, obj_s, bb_s = _loss_sums(tc2, tb2, gb2, to2, gc2, go2,
                                    interpret=interpret)
    inv_n = 1.0 / jnp.float32(n)
    cls_loss = cls_s[0, 0] * inv_n * 10000.0
    obj_loss = obj_s[0, 0] * inv_n * 5000.0
    bb_loss = bb_s[0, 0] * inv_n * 20000.0
    cls_loss = cls_loss * jnp.exp(-w_cls) + w_cls
    obj_loss = obj_loss * jnp.exp(-w_obj) + w_obj
    bb_loss = bb_loss * jnp.exp(-w_bb) + w_bb
    return (cls_loss, obj_loss, bb_loss)


# speedup vs baseline: 1.2697x; 1.2697x over previous
import functools

import jax
import jax.numpy as jnp
from jax.experimental import pallas as pl
from jax.experimental.pallas import tpu as pltpu

_ALPHA = 0.25
_DELTA = 0.1
_RB = 2048
_SUB = 128


def _focal(ce):
    p = jnp.exp(-ce)
    return _ALPHA * (1.0 - p) * (1.0 - p) * ce


def _loss_body(cls_r, tb_r, gb_r, to_r, gc_r, go_r, sa_r, sb_r, s4_r,
               cls_o, obj_o, bb_o):
    i = pl.program_id(0)

    @pl.when(i == 0)
    def _():
        cls_o[0, 0] = 0.0
        obj_o[0, 0] = 0.0
        bb_o[0, 0] = 0.0

    go = go_r[...]
    gc = gc_r[...]
    mask_obj = (go != -1).astype(jnp.float32)
    mask_bb = (go == 1).astype(jnp.float32)

    t = to_r[...]
    a = jax.lax.dot(t, sa_r[...])
    b = jax.lax.dot(t, sb_r[...])
    s2 = jnp.exp(a) + jnp.exp(b)
    sel2 = jnp.where(go == 1, b, a)
    ce2 = jnp.log(s2) - sel2
    obj_o[0, 0] += jnp.sum(_focal(ce2) * mask_obj)

    d = tb_r[...] - gb_r[...]
    ad = jnp.abs(d)
    sl1 = jnp.where(ad < _DELTA, (0.5 / _DELTA) * d * d, ad - 0.5 * _DELTA)
    bbsum = jax.lax.dot(sl1, s4_r[...])
    bb_o[0, 0] += jnp.sum(bbsum * mask_bb)

    x = cls_r[...]
    cls_ids = jax.lax.broadcasted_iota(jnp.int32, (80, _SUB), 0)
    acc = jnp.zeros((1, _SUB), dtype=jnp.float32)
    for j in range(_RB // _SUB):
        xt = jax.lax.transpose(x[j * _SUB:(j + 1) * _SUB, :], (1, 0))
        s = jnp.sum(jnp.exp(xt), axis=0, keepdims=True)
        oh = cls_ids == gc[j:j + 1, :]
        sel = jnp.sum(jnp.where(oh, xt, 0.0), axis=0, keepdims=True)
        ce = jnp.log(s) - sel
        acc += _focal(ce) * mask_bb[j:j + 1, :]
    cls_o[0, 0] += jnp.sum(acc)


@functools.partial(jax.jit, static_argnames=("interpret",))
def _loss_sums(cls2, tbf, gbf, tof, gcf, gof, interpret=False):
    n = cls2.shape[0]
    nb = n // _RB
    rows = _RB // _SUB

    k2 = jnp.arange(256, dtype=jnp.int32)[:, None]
    a2 = jnp.arange(128, dtype=jnp.int32)[None, :]
    sa = (k2 == 2 * a2).astype(jnp.float32)
    sb = (k2 == 2 * a2 + 1).astype(jnp.float32)
    k4 = jnp.arange(512, dtype=jnp.int32)[:, None]
    s4 = (k4 // 4 == a2).astype(jnp.float32)

    scalar_spec = pl.BlockSpec((1, 1), lambda i: (0, 0),
                               memory_space=pltpu.SMEM)
    return pl.pallas_call(
        _loss_body,
        grid=(nb,),
        in_specs=[
            pl.BlockSpec((_RB, 80), lambda i: (i, 0)),
            pl.BlockSpec((rows, 512), lambda i: (i, 0)),
            pl.BlockSpec((rows, 512), lambda i: (i, 0)),
            pl.BlockSpec((rows, 256), lambda i: (i, 0)),
            pl.BlockSpec((rows, 128), lambda i: (i, 0)),
            pl.BlockSpec((rows, 128), lambda i: (i, 0)),
            pl.BlockSpec((256, 128), lambda i: (0, 0)),
            pl.BlockSpec((256, 128), lambda i: (0, 0)),
            pl.BlockSpec((512, 128), lambda i: (0, 0)),
        ],
        out_specs=[scalar_spec, scalar_spec, scalar_spec],
        out_shape=[jax.ShapeDtypeStruct((1, 1), jnp.float32)] * 3,
        compiler_params=pltpu.CompilerParams(
            dimension_semantics=("arbitrary",)),
        interpret=interpret,
    )(cls2, tbf, gbf, tof, gcf, gof, sa, sb, s4)


def kernel(targets_bb, targets_cls, targets_obj, gt_targets_bb,
           gt_targets_cls, gt_targets_obj, w_obj, w_cls, w_bb, step,
           interpret=False):
    c = targets_cls.shape[-1]
    n = targets_cls.size // c
    cls2 = jnp.reshape(targets_cls, (n, c))
    tbf = jnp.reshape(targets_bb, (n * 4 // 512, 512))
    gbf = jnp.reshape(gt_targets_bb, (n * 4 // 512, 512))
    tof = jnp.reshape(targets_obj, (n * 2 // 256, 256))
    gcf = jnp.reshape(gt_targets_cls, (n // 128, 128))
    gof = jnp.reshape(gt_targets_obj, (n // 128, 128))

    cls_s, obj_s, bb_s = _loss_sums(cls2, tbf, gbf, tof, gcf, gof,
                                    interpret=interpret)
    inv_n = 1.0 / jnp.float32(n)
    cls_loss = cls_s[0, 0] * inv_n * 10000.0
    obj_loss = obj_s[0, 0] * inv_n * 5000.0
    bb_loss = bb_s[0, 0] * inv_n * 20000.0
    cls_loss = cls_loss * jnp.exp(-w_cls) + w_cls
    obj_loss = obj_loss * jnp.exp(-w_obj) + w_obj
    bb_loss = bb_loss * jnp.exp(-w_bb) + w_bb
    return (cls_loss, obj_loss, bb_loss)


# speedup vs baseline: 9.7710x; 7.6953x over previous
import functools

import jax
import jax.numpy as jnp
from jax.experimental import pallas as pl
from jax.experimental.pallas import tpu as pltpu

_ALPHA = 0.25
_DELTA = 0.1
_AB = 2048


def _focal(ce):
    p = jnp.exp(-ce)
    return _ALPHA * (1.0 - p) * (1.0 - p) * ce


def _loss_body(cls_r, tb_r, gb_r, to_r, gc_r, go_r, cls_o, obj_o, bb_o):
    i = pl.program_id(0)
    j = pl.program_id(1)

    @pl.when((i == 0) & (j == 0))
    def _():
        cls_o[0, 0] = 0.0
        obj_o[0, 0] = 0.0
        bb_o[0, 0] = 0.0

    go = go_r[0]
    gc = gc_r[0]
    mask_obj = (go != -1).astype(jnp.float32)
    mask_bb = (go == 1).astype(jnp.float32)

    x = cls_r[0]
    s = jnp.sum(jnp.exp(x), axis=0, keepdims=True)
    oh = jax.lax.broadcasted_iota(jnp.int32, x.shape, 0) == gc
    sel = jnp.sum(jnp.where(oh, x, 0.0), axis=0, keepdims=True)
    ce = jnp.log(s) - sel
    cls_o[0, 0] += jnp.sum(_focal(ce) * mask_bb)

    t = to_r[0]
    a = t[0:1, :]
    b = t[1:2, :]
    s2 = jnp.exp(a) + jnp.exp(b)
    sel2 = jnp.where(go == 1, b, a)
    ce2 = jnp.log(s2) - sel2
    obj_o[0, 0] += jnp.sum(_focal(ce2) * mask_obj)

    d = tb_r[0] - gb_r[0]
    ad = jnp.abs(d)
    sl1 = jnp.where(ad < _DELTA, (0.5 / _DELTA) * d * d, ad - 0.5 * _DELTA)
    bbsum = jnp.sum(sl1, axis=0, keepdims=True)
    bb_o[0, 0] += jnp.sum(bbsum * mask_bb)


@functools.partial(jax.jit, static_argnames=("interpret",))
def _loss_sums(clsT, tbT, gbT, toT, gc3, go3, interpret=False):
    bsz, c, a = clsT.shape
    nj = a // _AB
    scalar_spec = pl.BlockSpec((1, 1), lambda i, j: (0, 0),
                               memory_space=pltpu.SMEM)
    return pl.pallas_call(
        _loss_body,
        grid=(bsz, nj),
        in_specs=[
            pl.BlockSpec((1, c, _AB), lambda i, j: (i, 0, j)),
            pl.BlockSpec((1, 4, _AB), lambda i, j: (i, 0, j)),
            pl.BlockSpec((1, 4, _AB), lambda i, j: (i, 0, j)),
            pl.BlockSpec((1, 2, _AB), lambda i, j: (i, 0, j)),
            pl.BlockSpec((1, 1, _AB), lambda i, j: (i, 0, j)),
            pl.BlockSpec((1, 1, _AB), lambda i, j: (i, 0, j)),
        ],
        out_specs=[scalar_spec, scalar_spec, scalar_spec],
        out_shape=[jax.ShapeDtypeStruct((1, 1), jnp.float32)] * 3,
        compiler_params=pltpu.CompilerParams(
            dimension_semantics=("arbitrary", "arbitrary")),
        interpret=interpret,
    )(clsT, tbT, gbT, toT, gc3, go3)


def kernel(targets_bb, targets_cls, targets_obj, gt_targets_bb,
           gt_targets_cls, gt_targets_obj, w_obj, w_cls, w_bb, step,
           interpret=False):
    n = targets_cls.shape[0] * targets_cls.shape[1]
    clsT = jnp.transpose(targets_cls, (0, 2, 1))
    tbT = jnp.transpose(targets_bb, (0, 2, 1))
    gbT = jnp.transpose(gt_targets_bb, (0, 2, 1))
    toT = jnp.transpose(targets_obj, (0, 2, 1))
    gc3 = jnp.reshape(gt_targets_cls, (gt_targets_cls.shape[0], 1, -1))
    go3 = jnp.reshape(gt_targets_obj, (gt_targets_obj.shape[0], 1, -1))

    cls_s, obj_s, bb_s = _loss_sums(clsT, tbT, gbT, toT, gc3, go3,
                                    interpret=interpret)
    inv_n = 1.0 / jnp.float32(n)
    cls_loss = cls_s[0, 0] * inv_n * 10000.0
    obj_loss = obj_s[0, 0] * inv_n * 5000.0
    bb_loss = bb_s[0, 0] * inv_n * 20000.0
    cls_loss = cls_loss * jnp.exp(-w_cls) + w_cls
    obj_loss = obj_loss * jnp.exp(-w_obj) + w_obj
    bb_loss = bb_loss * jnp.exp(-w_bb) + w_bb
    return (cls_loss, obj_loss, bb_loss)


# speedup vs baseline: 22.8742x; 2.3410x over previous
import functools

import jax
import jax.numpy as jnp
from jax.experimental import pallas as pl
from jax.experimental.pallas import tpu as pltpu

_ALPHA = 0.25
_DELTA = 0.1
_AB = 2048


def _focal(ce):
    p = jnp.exp(-ce)
    return _ALPHA * (1.0 - p) * (1.0 - p) * ce


def _loss_body(cls_r, tb_r, gb_r, to_r, gc_r, go_r, cls_o, obj_o, bb_o):
    j = pl.program_id(0)

    @pl.when(j == 0)
    def _():
        cls_o[0, 0] = 0.0
        obj_o[0, 0] = 0.0
        bb_o[0, 0] = 0.0

    cls_acc = jnp.zeros((1, _AB), dtype=jnp.float32)
    obj_acc = jnp.zeros((1, _AB), dtype=jnp.float32)
    bb_acc = jnp.zeros((1, _AB), dtype=jnp.float32)
    for bi in range(cls_r.shape[0]):
        go = go_r[bi:bi + 1, :]
        gc = gc_r[bi:bi + 1, :]
        mask_obj = (go != -1).astype(jnp.float32)
        mask_bb = (go == 1).astype(jnp.float32)

        x = cls_r[bi]
        s = jnp.sum(jnp.exp(x), axis=0, keepdims=True)
        oh = jax.lax.broadcasted_iota(jnp.int32, x.shape, 0) == gc
        sel = jnp.sum(jnp.where(oh, x, 0.0), axis=0, keepdims=True)
        ce = jnp.log(s) - sel
        cls_acc += _focal(ce) * mask_bb

        t = to_r[bi]
        a = t[0:1, :]
        b = t[1:2, :]
        s2 = jnp.exp(a) + jnp.exp(b)
        sel2 = jnp.where(go == 1, b, a)
        ce2 = jnp.log(s2) - sel2
        obj_acc += _focal(ce2) * mask_obj

        d = tb_r[bi] - gb_r[bi]
        ad = jnp.abs(d)
        sl1 = jnp.where(ad < _DELTA, (0.5 / _DELTA) * d * d,
                        ad - 0.5 * _DELTA)
        bb_acc += jnp.sum(sl1, axis=0, keepdims=True) * mask_bb

    cls_o[0, 0] += jnp.sum(cls_acc)
    obj_o[0, 0] += jnp.sum(obj_acc)
    bb_o[0, 0] += jnp.sum(bb_acc)


@functools.partial(jax.jit, static_argnames=("interpret",))
def _loss_sums(clsT, tbT, gbT, toT, gc2, go2, interpret=False):
    bsz, c, a = clsT.shape
    nj = a // _AB
    scalar_spec = pl.BlockSpec((1, 1), lambda j: (0, 0),
                               memory_space=pltpu.SMEM)
    return pl.pallas_call(
        _loss_body,
        grid=(nj,),
        in_specs=[
            pl.BlockSpec((bsz, c, _AB), lambda j: (0, 0, j)),
            pl.BlockSpec((bsz, 4, _AB), lambda j: (0, 0, j)),
            pl.BlockSpec((bsz, 4, _AB), lambda j: (0, 0, j)),
            pl.BlockSpec((bsz, 2, _AB), lambda j: (0, 0, j)),
            pl.BlockSpec((bsz, _AB), lambda j: (0, j)),
            pl.BlockSpec((bsz, _AB), lambda j: (0, j)),
        ],
        out_specs=[scalar_spec, scalar_spec, scalar_spec],
        out_shape=[jax.ShapeDtypeStruct((1, 1), jnp.float32)] * 3,
        compiler_params=pltpu.CompilerParams(
            dimension_semantics=("arbitrary",)),
        interpret=interpret,
    )(clsT, tbT, gbT, toT, gc2, go2)


def kernel(targets_bb, targets_cls, targets_obj, gt_targets_bb,
           gt_targets_cls, gt_targets_obj, w_obj, w_cls, w_bb, step,
           interpret=False):
    n = targets_cls.shape[0] * targets_cls.shape[1]
    clsT = jnp.transpose(targets_cls, (0, 2, 1))
    tbT = jnp.transpose(targets_bb, (0, 2, 1))
    gbT = jnp.transpose(gt_targets_bb, (0, 2, 1))
    toT = jnp.transpose(targets_obj, (0, 2, 1))
    cls_s, obj_s, bb_s = _loss_sums(clsT, tbT, gbT, toT,
                                    gt_targets_cls, gt_targets_obj,
                                    interpret=interpret)
    inv_n = 1.0 / jnp.float32(n)
    cls_loss = cls_s[0, 0] * inv_n * 10000.0
    obj_loss = obj_s[0, 0] * inv_n * 5000.0
    bb_loss = bb_s[0, 0] * inv_n * 20000.0
    cls_loss = cls_loss * jnp.exp(-w_cls) + w_cls
    obj_loss = obj_loss * jnp.exp(-w_obj) + w_obj
    bb_loss = bb_loss * jnp.exp(-w_bb) + w_bb
    return (cls_loss, obj_loss, bb_loss)


# speedup vs baseline: 24.6781x; 1.0789x over previous
import functools

import jax
import jax.numpy as jnp
from jax.experimental import pallas as pl
from jax.experimental.pallas import tpu as pltpu

_ALPHA = 0.25
_DELTA = 0.1
_AB = 4096


def _focal(ce):
    p = jnp.exp(-ce)
    return _ALPHA * (1.0 - p) * (1.0 - p) * ce


def _loss_body(cls_r, tb_r, gb_r, to_r, gc_r, go_r, cls_o, obj_o, bb_o):
    j = pl.program_id(0)

    @pl.when(j == 0)
    def _():
        cls_o[0, 0] = 0.0
        obj_o[0, 0] = 0.0
        bb_o[0, 0] = 0.0

    cls_acc = jnp.zeros((1, _AB), dtype=jnp.float32)
    obj_acc = jnp.zeros((1, _AB), dtype=jnp.float32)
    bb_acc = jnp.zeros((1, _AB), dtype=jnp.float32)
    for bi in range(cls_r.shape[0]):
        go = go_r[bi:bi + 1, :]
        gc = gc_r[bi:bi + 1, :]
        mask_obj = (go != -1).astype(jnp.float32)
        mask_bb = (go == 1).astype(jnp.float32)

        x = cls_r[bi]
        s = jnp.sum(jnp.exp(x), axis=0, keepdims=True)
        oh = jax.lax.broadcasted_iota(jnp.int32, x.shape, 0) == gc
        sel = jnp.sum(jnp.where(oh, x, 0.0), axis=0, keepdims=True)
        ce = jnp.log(s) - sel
        cls_acc += _focal(ce) * mask_bb

        t = to_r[bi]
        a = t[0:1, :]
        b = t[1:2, :]
        s2 = jnp.exp(a) + jnp.exp(b)
        sel2 = jnp.where(go == 1, b, a)
        ce2 = jnp.log(s2) - sel2
        obj_acc += _focal(ce2) * mask_obj

        d = tb_r[bi] - gb_r[bi]
        ad = jnp.abs(d)
        sl1 = jnp.where(ad < _DELTA, (0.5 / _DELTA) * d * d,
                        ad - 0.5 * _DELTA)
        bb_acc += jnp.sum(sl1, axis=0, keepdims=True) * mask_bb

    cls_o[0, 0] += jnp.sum(cls_acc)
    obj_o[0, 0] += jnp.sum(obj_acc)
    bb_o[0, 0] += jnp.sum(bb_acc)


@functools.partial(jax.jit, static_argnames=("interpret",))
def _loss_sums(clsT, tbT, gbT, toT, gc2, go2, interpret=False):
    bsz, c, a = clsT.shape
    nj = a // _AB
    scalar_spec = pl.BlockSpec((1, 1), lambda j: (0, 0),
                               memory_space=pltpu.SMEM)
    return pl.pallas_call(
        _loss_body,
        grid=(nj,),
        in_specs=[
            pl.BlockSpec((bsz, c, _AB), lambda j: (0, 0, j)),
            pl.BlockSpec((bsz, 4, _AB), lambda j: (0, 0, j)),
            pl.BlockSpec((bsz, 4, _AB), lambda j: (0, 0, j)),
            pl.BlockSpec((bsz, 2, _AB), lambda j: (0, 0, j)),
            pl.BlockSpec((bsz, _AB), lambda j: (0, j)),
            pl.BlockSpec((bsz, _AB), lambda j: (0, j)),
        ],
        out_specs=[scalar_spec, scalar_spec, scalar_spec],
        out_shape=[jax.ShapeDtypeStruct((1, 1), jnp.float32)] * 3,
        compiler_params=pltpu.CompilerParams(
            dimension_semantics=("arbitrary",)),
        interpret=interpret,
    )(clsT, tbT, gbT, toT, gc2, go2)


def kernel(targets_bb, targets_cls, targets_obj, gt_targets_bb,
           gt_targets_cls, gt_targets_obj, w_obj, w_cls, w_bb, step,
           interpret=False):
    n = targets_cls.shape[0] * targets_cls.shape[1]
    clsT = jnp.transpose(targets_cls, (0, 2, 1))
    tbT = jnp.transpose(targets_bb, (0, 2, 1))
    gbT = jnp.transpose(gt_targets_bb, (0, 2, 1))
    toT = jnp.transpose(targets_obj, (0, 2, 1))
    cls_s, obj_s, bb_s = _loss_sums(clsT, tbT, gbT, toT,
                                    gt_targets_cls, gt_targets_obj,
                                    interpret=interpret)
    inv_n = 1.0 / jnp.float32(n)
    cls_loss = cls_s[0, 0] * inv_n * 10000.0
    obj_loss = obj_s[0, 0] * inv_n * 5000.0
    bb_loss = bb_s[0, 0] * inv_n * 20000.0
    cls_loss = cls_loss * jnp.exp(-w_cls) + w_cls
    obj_loss = obj_loss * jnp.exp(-w_obj) + w_obj
    bb_loss = bb_loss * jnp.exp(-w_bb) + w_bb
    return (cls_loss, obj_loss, bb_loss)
